# baseline (speedup 1.0000x reference)
import functools

import jax
import jax.numpy as jnp
from jax import lax
from jax.experimental import pallas as pl
from jax.experimental.pallas import tpu as pltpu
from jax.experimental.pallas import tpu_sc as plsc

B = 16384
IMG_IN = 512
DU = 64
DI = 32
NC = 2
NS = 16
NW = NC * NS
BPW = B // NW
CHUNK = 128
NCH = BPW // CHUNK

BB = 512
GRID = B // BB


def _sc_gather(user_idx, item_idx, uf, itf, ub_flat, ib_flat):
    mesh = plsc.VectorSubcoreMesh(core_axis_name="c", subcore_axis_name="s")

    @functools.partial(
        pl.kernel,
        out_type=(
            jax.ShapeDtypeStruct((B, DU), jnp.float32),
            jax.ShapeDtypeStruct((B, DI), jnp.float32),
            jax.ShapeDtypeStruct((B,), jnp.float32),
        ),
        mesh=mesh,
        compiler_params=pltpu.CompilerParams(use_tc_tiling_on_sc=False),
        scratch_types=(
            pltpu.VMEM((BPW,), jnp.int32),
            pltpu.VMEM((BPW,), jnp.int32),
            pltpu.VMEM((BPW, DU), jnp.float32),
            pltpu.VMEM((BPW, DI), jnp.float32),
            pltpu.VMEM((BPW,), jnp.float32),
            pltpu.VMEM((BPW,), jnp.float32),
            pltpu.VMEM((BPW,), jnp.float32),
            pltpu.SemaphoreType.DMA,
        ),
    )
    def k(uidx_hbm, iidx_hbm, uf_hbm, if_hbm, ub_hbm, ib_hbm,
          urows_out, irows_out, bias_out,
          uidx_v, iidx_v, urows_v, irows_v, ub_v, ib_v, bsum_v, sem):
        wid = lax.axis_index("s") * NC + lax.axis_index("c")
        base = wid * BPW
        pltpu.sync_copy(uidx_hbm.at[pl.ds(base, BPW)], uidx_v)
        pltpu.sync_copy(iidx_hbm.at[pl.ds(base, BPW)], iidx_v)
        copies = []
        for c in range(NCH):
            sl = pl.ds(c * CHUNK, CHUNK)
            copies.append(pltpu.async_copy(uf_hbm.at[uidx_v.at[sl]], urows_v.at[sl], sem))
            copies.append(pltpu.async_copy(if_hbm.at[iidx_v.at[sl]], irows_v.at[sl], sem))
            copies.append(pltpu.async_copy(ub_hbm.at[uidx_v.at[sl]], ub_v.at[sl], sem))
            copies.append(pltpu.async_copy(ib_hbm.at[iidx_v.at[sl]], ib_v.at[sl], sem))
        for cp in copies:
            cp.wait()
        for i in range(BPW // 16):
            sl = pl.ds(i * 16, 16)
            bsum_v[sl] = ub_v[sl] + ib_v[sl]
        pltpu.sync_copy(urows_v, urows_out.at[pl.ds(base, BPW)])
        pltpu.sync_copy(irows_v, irows_out.at[pl.ds(base, BPW)])
        pltpu.sync_copy(bsum_v, bias_out.at[pl.ds(base, BPW)])

    return k(user_idx, item_idx, uf, itf, ub_flat, ib_flat)


def _tc_body(img_ref, w_ref, b_ref, u_ref, it_ref, bs_ref, o_ref):
    img = jnp.dot(img_ref[...], w_ref[...], preferred_element_type=jnp.float32)
    img = img + b_ref[...]
    u = u_ref[...]
    t = u[:, :DI] * img + u[:, DI:] * it_ref[...]
    o_ref[...] = jnp.sum(t, axis=1) + bs_ref[...]


def kernel(image, user, item, user_factors, item_factors, user_biases,
           item_biases, W_img, b_img):
    user = user.astype(jnp.int32)
    item = item.astype(jnp.int32)
    urows, irows, bsum = _sc_gather(
        user, item, user_factors, item_factors,
        user_biases.reshape(-1), item_biases.reshape(-1))
    out = pl.pallas_call(
        _tc_body,
        grid=(GRID,),
        in_specs=[
            pl.BlockSpec((BB, IMG_IN), lambda i: (i, 0)),
            pl.BlockSpec((IMG_IN, DI), lambda i: (0, 0)),
            pl.BlockSpec((1, DI), lambda i: (0, 0)),
            pl.BlockSpec((BB, DU), lambda i: (i, 0)),
            pl.BlockSpec((BB, DI), lambda i: (i, 0)),
            pl.BlockSpec((BB,), lambda i: (i,)),
        ],
        out_specs=pl.BlockSpec((BB,), lambda i: (i,)),
        out_shape=jax.ShapeDtypeStruct((B,), jnp.float32),
    )(image, W_img, b_img.reshape(1, DI), urows, irows, bsum)
    return out
